# Initial kernel scaffold; baseline (speedup 1.0000x reference)
import jax
import jax.numpy as jnp
from jax.experimental import pallas as pl


def kernel(x, edge_index, batch, descriptors, W1, b1, g1, be1, Wb_all, Wc_all, cb_all, ng_all, nb_all, Wm1, gm1, bm1, Wm2, gm2, bm2, W2, b2, g2, be2, Wo, bo):
    raise NotImplementedError("write your pallas kernel here")


# speedup vs baseline: 6.7602x; 6.7602x over previous
import functools

import jax
import jax.numpy as jnp
from jax import lax
from jax.experimental import pallas as pl
from jax.experimental.pallas import tpu as pltpu
from jax.experimental.pallas import tpu_sc as plsc

N = 50000
E = 800000
G = 512
HID = 128
LAYERS = 4
HEADS = 8
BASES = 4
DLEN = 200
FIN = 27

NC = 2
NS = 16
CH = 128
NCH_E = E // CH
NACC = 50048
STRIPE = NACC // NS
BLK = 2000
GRID = N // BLK
NPOOL = NACC
NCH_P = NPOOL // CH
PROWS = 640
PSTRIPE = PROWS // NS

_mesh = plsc.VectorSubcoreMesh(
    core_axis_name="c", subcore_axis_name="s", num_cores=NC, num_subcores=NS)
_sc_params = pltpu.CompilerParams(use_tc_tiling_on_sc=False)


@functools.partial(
    pl.kernel,
    out_type=(jax.ShapeDtypeStruct((NACC, 8), jnp.float32),
              jax.ShapeDtypeStruct((NACC, 8), jnp.float32)),
    mesh=_mesh,
    compiler_params=_sc_params,
    scratch_types=[
        pltpu.VMEM((CH,), jnp.int32),
        pltpu.VMEM((CH, 8), jnp.float32),
        pltpu.VMEM_SHARED((NACC, 8), jnp.float32),
    ],
)
def _sc_degree(col_h, ones_h, z_h, d0_h, d1_h, colv, onesv, acc):
    c = lax.axis_index("c")
    s = lax.axis_index("s")
    wid = s * NC + c
    pltpu.sync_copy(z_h, acc.at[pl.ds(s * STRIPE, STRIPE)])
    pltpu.sync_copy(ones_h, onesv)
    plsc.subcore_barrier()

    def step(g, carry):
        chunk = g * (NC * NS) + wid

        @pl.when(chunk < NCH_E)
        def _():
            pltpu.sync_copy(col_h.at[pl.ds(chunk * CH, CH)], colv)
            pltpu.sync_copy(onesv, acc.at[colv], add=True)

        return carry

    lax.fori_loop(0, (NCH_E + NC * NS - 1) // (NC * NS), step, 0)
    plsc.subcore_barrier()
    sl = pl.ds(s * STRIPE, STRIPE)

    @pl.when(c == 0)
    def _():
        pltpu.sync_copy(acc.at[sl], d0_h.at[sl])

    @pl.when(c == 1)
    def _():
        pltpu.sync_copy(acc.at[sl], d1_h.at[sl])


@functools.partial(
    pl.kernel,
    out_type=(jax.ShapeDtypeStruct((NACC, 32), jnp.float32),
              jax.ShapeDtypeStruct((NACC, 32), jnp.float32)),
    mesh=_mesh,
    compiler_params=_sc_params,
    scratch_types=[
        pltpu.VMEM((CH,), jnp.int32),
        pltpu.VMEM((CH,), jnp.int32),
        pltpu.VMEM((CH, 32), jnp.float32),
        pltpu.VMEM_SHARED((NACC, 32), jnp.float32),
        pltpu.SemaphoreType.DMA,
    ],
)
def _sc_edge_agg(row_h, col_h, tlo_h, thi_h, z_h, olo_h, ohi_h,
                 rowv, colv, rowsv, acc, sem):
    c = lax.axis_index("c")
    s = lax.axis_index("s")
    pltpu.sync_copy(z_h, acc.at[pl.ds(s * STRIPE, STRIPE)])
    plsc.subcore_barrier()

    def step(g, carry):
        chunk = g * NS + s

        @pl.when(chunk < NCH_E)
        def _():
            base = chunk * CH
            pltpu.sync_copy(row_h.at[pl.ds(base, CH)], rowv)
            pltpu.sync_copy(col_h.at[pl.ds(base, CH)], colv)

            @pl.when(c == 0)
            def _():
                pltpu.async_copy(tlo_h.at[rowv], rowsv, sem).wait()

            @pl.when(c == 1)
            def _():
                pltpu.async_copy(thi_h.at[rowv], rowsv, sem).wait()

            pltpu.sync_copy(rowsv, acc.at[colv], add=True)

        return carry

    lax.fori_loop(0, (NCH_E + NS - 1) // NS, step, 0)
    plsc.subcore_barrier()
    sl = pl.ds(s * STRIPE, STRIPE)

    @pl.when(c == 0)
    def _():
        pltpu.sync_copy(acc.at[sl], olo_h.at[sl])

    @pl.when(c == 1)
    def _():
        pltpu.sync_copy(acc.at[sl], ohi_h.at[sl])


@functools.partial(
    pl.kernel,
    out_type=(jax.ShapeDtypeStruct((PROWS, HID), jnp.float32),
              jax.ShapeDtypeStruct((PROWS, HID), jnp.float32),
              jax.ShapeDtypeStruct((PROWS, 8), jnp.float32),
              jax.ShapeDtypeStruct((PROWS, 8), jnp.float32)),
    mesh=_mesh,
    compiler_params=_sc_params,
    scratch_types=[
        pltpu.VMEM((CH,), jnp.int32),
        pltpu.VMEM((CH, HID), jnp.float32),
        pltpu.VMEM((CH, 8), jnp.float32),
        pltpu.VMEM_SHARED((PROWS, HID), jnp.float32),
        pltpu.VMEM_SHARED((PROWS, 8), jnp.float32),
    ],
)
def _sc_pool(h_h, batch_h, ones_h, zp_h, zc_h, p0_h, p1_h, c0_h, c1_h,
             bv, hv, onesv, accp, accc):
    c = lax.axis_index("c")
    s = lax.axis_index("s")
    wid = s * NC + c
    pltpu.sync_copy(zp_h, accp.at[pl.ds(s * PSTRIPE, PSTRIPE)])
    pltpu.sync_copy(zc_h, accc.at[pl.ds(s * PSTRIPE, PSTRIPE)])
    pltpu.sync_copy(ones_h, onesv)
    plsc.subcore_barrier()

    def step(g, carry):
        chunk = g * (NC * NS) + wid

        @pl.when(chunk < NCH_P)
        def _():
            base = chunk * CH
            pltpu.sync_copy(batch_h.at[pl.ds(base, CH)], bv)
            pltpu.sync_copy(h_h.at[pl.ds(base, CH)], hv)
            pltpu.sync_copy(hv, accp.at[bv], add=True)
            pltpu.sync_copy(onesv, accc.at[bv], add=True)

        return carry

    lax.fori_loop(0, (NCH_P + NC * NS - 1) // (NC * NS), step, 0)
    plsc.subcore_barrier()
    sl = pl.ds(s * PSTRIPE, PSTRIPE)

    @pl.when(c == 0)
    def _():
        pltpu.sync_copy(accp.at[sl], p0_h.at[sl])
        pltpu.sync_copy(accc.at[sl], c0_h.at[sl])

    @pl.when(c == 1)
    def _():
        pltpu.sync_copy(accp.at[sl], p1_h.at[sl])
        pltpu.sync_copy(accc.at[sl], c1_h.at[sl])


def _acc_stats(y, s_ref, ss_ref):
    ps = jnp.sum(y, axis=0, keepdims=True)
    pss = jnp.sum(y * y, axis=0, keepdims=True)

    @pl.when(pl.program_id(0) == 0)
    def _():
        s_ref[...] = ps
        ss_ref[...] = pss

    @pl.when(pl.program_id(0) > 0)
    def _():
        s_ref[...] += ps
        ss_ref[...] += pss


def _lin1_body(x_ref, w_ref, b_ref, y_ref, s_ref, ss_ref):
    y = jnp.dot(x_ref[...], w_ref[...],
                preferred_element_type=jnp.float32) + b_ref[...]
    y_ref[...] = y
    _acc_stats(y, s_ref, ss_ref)


def _bn_body(y_ref, s_ref, ss_ref, g_ref, b_ref, h_ref):
    mu = s_ref[...] / N
    var = ss_ref[...] / N - mu * mu
    inv = lax.rsqrt(var + 1e-5)
    h_ref[...] = jnp.maximum((y_ref[...] - mu) * inv * g_ref[...] + b_ref[...],
                             0.0)


def _bn_res_body(y_ref, s_ref, ss_ref, g_ref, b_ref, r_ref, h_ref):
    mu = s_ref[...] / N
    var = ss_ref[...] / N - mu * mu
    inv = lax.rsqrt(var + 1e-5)
    h_ref[...] = r_ref[...] + jnp.maximum(
        (y_ref[...] - mu) * inv * g_ref[...] + b_ref[...], 0.0)


def _bc_body(h_ref, wb_ref, wc_ref, cb_ref, d0_ref, d1_ref,
             lo_ref, hi_ref, comb_ref):
    dis = lax.rsqrt(d0_ref[:, 0:1] + d1_ref[:, 0:1] + 1.0)
    bp = dis * jnp.dot(h_ref[...], wb_ref[...],
                       preferred_element_type=jnp.float32)
    lo_ref[...] = bp[:, :32]
    hi_ref[...] = bp[:, 32:]
    comb_ref[...] = jnp.dot(h_ref[...], wc_ref[...],
                            preferred_element_type=jnp.float32) + cb_ref[...]


def _mix_body(alo_ref, ahi_ref, blo_ref, bhi_ref, comb_ref, d0_ref, d1_ref,
              y_ref, s_ref, ss_ref):
    dis = lax.rsqrt(d0_ref[:, 0:1] + d1_ref[:, 0:1] + 1.0)
    alo = dis * (alo_ref[...] + blo_ref[...])
    ahi = dis * (ahi_ref[...] + bhi_ref[...])
    a = (alo[:, :16], alo[:, 16:], ahi[:, :16], ahi[:, 16:])
    comb = comb_ref[...]
    pieces = []
    for hh in range(HEADS):
        p = comb[:, hh * BASES:hh * BASES + 1] * a[0]
        for b in range(1, BASES):
            p = p + comb[:, hh * BASES + b:hh * BASES + b + 1] * a[b]
        pieces.append(p)
    y = jnp.concatenate(pieces, axis=1)
    y_ref[...] = y
    _acc_stats(y, s_ref, ss_ref)


def _head_body(p0_ref, p1_ref, c0_ref, c1_ref, d_ref, wm1_ref, gm1_ref,
               bm1_ref, wm2_ref, gm2_ref, bm2_ref, w2m_ref, w2d_ref, b2_ref,
               g2_ref, be2_ref, wo_ref, bo_ref, o_ref):
    def bn(t, g, b):
        mu = jnp.mean(t, axis=0, keepdims=True)
        var = jnp.mean(t * t, axis=0, keepdims=True) - mu * mu
        return (t - mu) * lax.rsqrt(var + 1e-5) * g + b

    cnt = c0_ref[:G, 0:1] + c1_ref[:G, 0:1]
    pooled = (p0_ref[:G, :] + p1_ref[:G, :]) / jnp.maximum(cnt, 1.0)
    m = jnp.maximum(bn(jnp.dot(pooled, wm1_ref[...],
                               preferred_element_type=jnp.float32),
                       gm1_ref[...], bm1_ref[...]), 0.0)
    m = jnp.maximum(bn(jnp.dot(m, wm2_ref[...],
                               preferred_element_type=jnp.float32),
                       gm2_ref[...], bm2_ref[...]), 0.0)
    z = (jnp.dot(m, w2m_ref[...], preferred_element_type=jnp.float32) +
         jnp.dot(d_ref[...], w2d_ref[...], preferred_element_type=jnp.float32)
         + b2_ref[...])
    z = jnp.maximum(bn(z, g2_ref[...], be2_ref[...]), 0.0)
    o_ref[...] = jnp.dot(z, wo_ref[...],
                         preferred_element_type=jnp.float32) + bo_ref[...]


def _row_spec(w):
    return pl.BlockSpec((BLK, w), lambda i: (i, 0))


def _full(a):
    return pl.BlockSpec(a.shape, lambda i: tuple(0 for _ in a.shape))


_STAT_SPEC = pl.BlockSpec((1, HID), lambda i: (0, 0))


def kernel(x, edge_index, batch, descriptors, W1, b1, g1, be1, Wb_all, Wc_all,
           cb_all, ng_all, nb_all, Wm1, gm1, bm1, Wm2, gm2, bm2, W2, b2, g2,
           be2, Wo, bo):
    f32 = jnp.float32
    row = edge_index[0]
    col = edge_index[1]
    ones8 = jnp.ones((CH, 8), f32)
    z8 = jnp.zeros((STRIPE, 8), f32)
    z32 = jnp.zeros((STRIPE, 32), f32)
    zp = jnp.zeros((PSTRIPE, HID), f32)
    zc = jnp.zeros((PSTRIPE, 8), f32)
    batchp = jnp.concatenate(
        [batch, jnp.full((NPOOL - N,), G, jnp.int32)])

    d0, d1 = _sc_degree(col, ones8, z8)

    y, s, ss = pl.pallas_call(
        _lin1_body,
        grid=(GRID,),
        in_specs=[pl.BlockSpec((BLK, FIN), lambda i: (i, 0)),
                  _full(W1), pl.BlockSpec((1, HID), lambda i: (0, 0))],
        out_specs=[_row_spec(HID), _STAT_SPEC, _STAT_SPEC],
        out_shape=[jax.ShapeDtypeStruct((NPOOL, HID), f32),
                   jax.ShapeDtypeStruct((1, HID), f32),
                   jax.ShapeDtypeStruct((1, HID), f32)],
    )(x, W1, b1.reshape(1, HID))
    h = pl.pallas_call(
        _bn_body,
        grid=(GRID,),
        in_specs=[_row_spec(HID), _STAT_SPEC, _STAT_SPEC,
                  pl.BlockSpec((1, HID), lambda i: (0, 0)),
                  pl.BlockSpec((1, HID), lambda i: (0, 0))],
        out_specs=_row_spec(HID),
        out_shape=jax.ShapeDtypeStruct((NPOOL, HID), f32),
    )(y, s, ss, g1.reshape(1, HID), be1.reshape(1, HID))

    for l in range(LAYERS):
        lo, hi, comb = pl.pallas_call(
            _bc_body,
            grid=(GRID,),
            in_specs=[_row_spec(HID), _full(Wb_all[l]), _full(Wc_all[l]),
                      pl.BlockSpec((1, 32), lambda i: (0, 0)),
                      _row_spec(8), _row_spec(8)],
            out_specs=[_row_spec(32), _row_spec(32), _row_spec(32)],
            out_shape=[jax.ShapeDtypeStruct((N, 32), f32),
                       jax.ShapeDtypeStruct((N, 32), f32),
                       jax.ShapeDtypeStruct((N, 32), f32)],
        )(h, Wb_all[l], Wc_all[l], cb_all[l].reshape(1, 32), d0, d1)

        acclo, acchi = _sc_edge_agg(row, col, lo, hi, z32)

        y, s, ss = pl.pallas_call(
            _mix_body,
            grid=(GRID,),
            in_specs=[_row_spec(32), _row_spec(32), _row_spec(32),
                      _row_spec(32), _row_spec(32), _row_spec(8),
                      _row_spec(8)],
            out_specs=[_row_spec(HID), _STAT_SPEC, _STAT_SPEC],
            out_shape=[jax.ShapeDtypeStruct((NPOOL, HID), f32),
                       jax.ShapeDtypeStruct((1, HID), f32),
                       jax.ShapeDtypeStruct((1, HID), f32)],
        )(acclo, acchi, lo, hi, comb, d0, d1)

        h = pl.pallas_call(
            _bn_res_body,
            grid=(GRID,),
            in_specs=[_row_spec(HID), _STAT_SPEC, _STAT_SPEC,
                      pl.BlockSpec((1, HID), lambda i: (0, 0)),
                      pl.BlockSpec((1, HID), lambda i: (0, 0)),
                      _row_spec(HID)],
            out_specs=_row_spec(HID),
            out_shape=jax.ShapeDtypeStruct((NPOOL, HID), f32),
        )(y, s, ss, ng_all[l].reshape(1, HID), nb_all[l].reshape(1, HID), h)

    p0, p1, c0, c1 = _sc_pool(h, batchp, ones8, zp, zc)

    out = pl.pallas_call(
        _head_body,
        out_shape=jax.ShapeDtypeStruct((G, 1), f32),
    )(p0, p1, c0, c1, descriptors, Wm1, gm1.reshape(1, 64),
      bm1.reshape(1, 64), Wm2, gm2.reshape(1, 32), bm2.reshape(1, 32),
      W2[:32], W2[32:], b2.reshape(1, HID), g2.reshape(1, HID),
      be2.reshape(1, HID), Wo, bo.reshape(1, 1))
    return out


# speedup vs baseline: 8.4941x; 1.2565x over previous
import functools

import jax
import jax.numpy as jnp
from jax import lax
from jax.experimental import pallas as pl
from jax.experimental.pallas import tpu as pltpu
from jax.experimental.pallas import tpu_sc as plsc

N = 50000
E = 800000
G = 512
HID = 128
LAYERS = 4
HEADS = 8
BASES = 4
DLEN = 200
FIN = 27

NC = 2
NS = 16
CH = 128
NCH_E = E // CH
NACC = 50048
STRIPE = NACC // NS
BLK = 2000
GRID = N // BLK
NPOOL = NACC
NCH_P = NPOOL // CH
PROWS = 640
PSTRIPE = PROWS // NS

_mesh = plsc.VectorSubcoreMesh(
    core_axis_name="c", subcore_axis_name="s", num_cores=NC, num_subcores=NS)
_sc_params = pltpu.CompilerParams(use_tc_tiling_on_sc=False)


@functools.partial(
    pl.kernel,
    out_type=(jax.ShapeDtypeStruct((NACC, 8), jnp.float32),
              jax.ShapeDtypeStruct((NACC, 8), jnp.float32)),
    mesh=_mesh,
    compiler_params=_sc_params,
    scratch_types=[
        pltpu.VMEM((CH,), jnp.int32),
        pltpu.VMEM((CH, 8), jnp.float32),
        pltpu.VMEM_SHARED((NACC, 8), jnp.float32),
    ],
)
def _sc_degree(col_h, ones_h, z_h, d0_h, d1_h, colv, onesv, acc):
    c = lax.axis_index("c")
    s = lax.axis_index("s")
    wid = s * NC + c
    pltpu.sync_copy(z_h, acc.at[pl.ds(s * STRIPE, STRIPE)])
    pltpu.sync_copy(ones_h, onesv)
    plsc.subcore_barrier()

    def step(g, carry):
        chunk = g * (NC * NS) + wid

        @pl.when(chunk < NCH_E)
        def _():
            pltpu.sync_copy(col_h.at[pl.ds(chunk * CH, CH)], colv)
            pltpu.sync_copy(onesv, acc.at[colv], add=True)

        return carry

    lax.fori_loop(0, (NCH_E + NC * NS - 1) // (NC * NS), step, 0)
    plsc.subcore_barrier()
    sl = pl.ds(s * STRIPE, STRIPE)

    @pl.when(c == 0)
    def _():
        pltpu.sync_copy(acc.at[sl], d0_h.at[sl])

    @pl.when(c == 1)
    def _():
        pltpu.sync_copy(acc.at[sl], d1_h.at[sl])


@functools.partial(
    pl.kernel,
    out_type=(jax.ShapeDtypeStruct((NACC, 32), jnp.float32),
              jax.ShapeDtypeStruct((NACC, 32), jnp.float32)),
    mesh=_mesh,
    compiler_params=_sc_params,
    scratch_types=[
        pltpu.VMEM((CH,), jnp.int32),
        pltpu.VMEM((CH,), jnp.int32),
        pltpu.VMEM((CH,), jnp.int32),
        pltpu.VMEM((CH,), jnp.int32),
        pltpu.VMEM((CH, 32), jnp.float32),
        pltpu.VMEM((CH, 32), jnp.float32),
        pltpu.VMEM_SHARED((NACC, 32), jnp.float32),
        pltpu.SemaphoreType.DMA,
        pltpu.SemaphoreType.DMA,
    ],
)
def _sc_edge_agg(row_h, col_h, tlo_h, thi_h, z_h, olo_h, ohi_h,
                 rowv0, rowv1, colv0, colv1, rows0, rows1, acc, sem0, sem1):
    c = lax.axis_index("c")
    s = lax.axis_index("s")
    pltpu.sync_copy(z_h, acc.at[pl.ds(s * STRIPE, STRIPE)])
    plsc.subcore_barrier()

    def fire(g, rowv, colv, rowsv, sem):
        chunk = g * NS + s

        @pl.when(chunk < NCH_E)
        def _():
            base = chunk * CH
            pltpu.sync_copy(row_h.at[pl.ds(base, CH)], rowv)
            pltpu.sync_copy(col_h.at[pl.ds(base, CH)], colv)

            @pl.when(c == 0)
            def _():
                pltpu.async_copy(tlo_h.at[rowv], rowsv, sem)

            @pl.when(c == 1)
            def _():
                pltpu.async_copy(thi_h.at[rowv], rowsv, sem)

    def drain(g, rowv, colv, rowsv, sem):
        chunk = g * NS + s

        @pl.when(chunk < NCH_E)
        def _():
            @pl.when(c == 0)
            def _():
                pltpu.make_async_copy(tlo_h.at[rowv], rowsv, sem).wait()

            @pl.when(c == 1)
            def _():
                pltpu.make_async_copy(thi_h.at[rowv], rowsv, sem).wait()

            pltpu.sync_copy(rowsv, acc.at[colv], add=True)

    fire(0, rowv0, colv0, rows0, sem0)

    def step(g2, carry):
        g0 = g2 * 2
        fire(g0 + 1, rowv1, colv1, rows1, sem1)
        drain(g0, rowv0, colv0, rows0, sem0)
        fire(g0 + 2, rowv0, colv0, rows0, sem0)
        drain(g0 + 1, rowv1, colv1, rows1, sem1)
        return carry

    lax.fori_loop(0, ((NCH_E + NS - 1) // NS + 1) // 2, step, 0)
    plsc.subcore_barrier()
    sl = pl.ds(s * STRIPE, STRIPE)

    @pl.when(c == 0)
    def _():
        pltpu.sync_copy(acc.at[sl], olo_h.at[sl])

    @pl.when(c == 1)
    def _():
        pltpu.sync_copy(acc.at[sl], ohi_h.at[sl])


@functools.partial(
    pl.kernel,
    out_type=(jax.ShapeDtypeStruct((PROWS, HID), jnp.float32),
              jax.ShapeDtypeStruct((PROWS, HID), jnp.float32),
              jax.ShapeDtypeStruct((PROWS, 8), jnp.float32),
              jax.ShapeDtypeStruct((PROWS, 8), jnp.float32)),
    mesh=_mesh,
    compiler_params=_sc_params,
    scratch_types=[
        pltpu.VMEM((CH,), jnp.int32),
        pltpu.VMEM((CH, HID), jnp.float32),
        pltpu.VMEM((CH, 8), jnp.float32),
        pltpu.VMEM_SHARED((PROWS, HID), jnp.float32),
        pltpu.VMEM_SHARED((PROWS, 8), jnp.float32),
    ],
)
def _sc_pool(h_h, batch_h, ones_h, zp_h, zc_h, p0_h, p1_h, c0_h, c1_h,
             bv, hv, onesv, accp, accc):
    c = lax.axis_index("c")
    s = lax.axis_index("s")
    wid = s * NC + c
    pltpu.sync_copy(zp_h, accp.at[pl.ds(s * PSTRIPE, PSTRIPE)])
    pltpu.sync_copy(zc_h, accc.at[pl.ds(s * PSTRIPE, PSTRIPE)])
    pltpu.sync_copy(ones_h, onesv)
    plsc.subcore_barrier()

    def step(g, carry):
        chunk = g * (NC * NS) + wid

        @pl.when(chunk < NCH_P)
        def _():
            base = chunk * CH
            pltpu.sync_copy(batch_h.at[pl.ds(base, CH)], bv)
            pltpu.sync_copy(h_h.at[pl.ds(base, CH)], hv)
            pltpu.sync_copy(hv, accp.at[bv], add=True)
            pltpu.sync_copy(onesv, accc.at[bv], add=True)

        return carry

    lax.fori_loop(0, (NCH_P + NC * NS - 1) // (NC * NS), step, 0)
    plsc.subcore_barrier()
    sl = pl.ds(s * PSTRIPE, PSTRIPE)

    @pl.when(c == 0)
    def _():
        pltpu.sync_copy(accp.at[sl], p0_h.at[sl])
        pltpu.sync_copy(accc.at[sl], c0_h.at[sl])

    @pl.when(c == 1)
    def _():
        pltpu.sync_copy(accp.at[sl], p1_h.at[sl])
        pltpu.sync_copy(accc.at[sl], c1_h.at[sl])


def _acc_stats(y, s_ref, ss_ref):
    ps = jnp.sum(y, axis=0, keepdims=True)
    pss = jnp.sum(y * y, axis=0, keepdims=True)

    @pl.when(pl.program_id(0) == 0)
    def _():
        s_ref[...] = ps
        ss_ref[...] = pss

    @pl.when(pl.program_id(0) > 0)
    def _():
        s_ref[...] += ps
        ss_ref[...] += pss


def _lin1_body(x_ref, w_ref, b_ref, y_ref, s_ref, ss_ref):
    y = jnp.dot(x_ref[...], w_ref[...],
                preferred_element_type=jnp.float32) + b_ref[...]
    y_ref[...] = y
    _acc_stats(y, s_ref, ss_ref)


def _bn_body(y_ref, s_ref, ss_ref, g_ref, b_ref, h_ref):
    mu = s_ref[...] / N
    var = ss_ref[...] / N - mu * mu
    inv = lax.rsqrt(var + 1e-5)
    h_ref[...] = jnp.maximum((y_ref[...] - mu) * inv * g_ref[...] + b_ref[...],
                             0.0)


def _bn_res_body(y_ref, s_ref, ss_ref, g_ref, b_ref, r_ref, h_ref):
    mu = s_ref[...] / N
    var = ss_ref[...] / N - mu * mu
    inv = lax.rsqrt(var + 1e-5)
    h_ref[...] = r_ref[...] + jnp.maximum(
        (y_ref[...] - mu) * inv * g_ref[...] + b_ref[...], 0.0)


def _bc_body(h_ref, wb_ref, wc_ref, cb_ref, d0_ref, d1_ref,
             lo_ref, hi_ref, comb_ref):
    dis = lax.rsqrt(d0_ref[:, 0:1] + d1_ref[:, 0:1] + 1.0)
    bp = dis * jnp.dot(h_ref[...], wb_ref[...],
                       preferred_element_type=jnp.float32)
    lo_ref[...] = bp[:, :32]
    hi_ref[...] = bp[:, 32:]
    comb_ref[...] = jnp.dot(h_ref[...], wc_ref[...],
                            preferred_element_type=jnp.float32) + cb_ref[...]


def _mix_body(alo_ref, ahi_ref, blo_ref, bhi_ref, comb_ref, d0_ref, d1_ref,
              y_ref, s_ref, ss_ref):
    dis = lax.rsqrt(d0_ref[:, 0:1] + d1_ref[:, 0:1] + 1.0)
    alo = dis * (alo_ref[...] + blo_ref[...])
    ahi = dis * (ahi_ref[...] + bhi_ref[...])
    a = (alo[:, :16], alo[:, 16:], ahi[:, :16], ahi[:, 16:])
    comb = comb_ref[...]
    pieces = []
    for hh in range(HEADS):
        p = comb[:, hh * BASES:hh * BASES + 1] * a[0]
        for b in range(1, BASES):
            p = p + comb[:, hh * BASES + b:hh * BASES + b + 1] * a[b]
        pieces.append(p)
    y = jnp.concatenate(pieces, axis=1)
    y_ref[...] = y
    _acc_stats(y, s_ref, ss_ref)


def _head_body(p0_ref, p1_ref, c0_ref, c1_ref, d_ref, wm1_ref, gm1_ref,
               bm1_ref, wm2_ref, gm2_ref, bm2_ref, w2m_ref, w2d_ref, b2_ref,
               g2_ref, be2_ref, wo_ref, bo_ref, o_ref):
    def bn(t, g, b):
        mu = jnp.mean(t, axis=0, keepdims=True)
        var = jnp.mean(t * t, axis=0, keepdims=True) - mu * mu
        return (t - mu) * lax.rsqrt(var + 1e-5) * g + b

    cnt = c0_ref[:G, 0:1] + c1_ref[:G, 0:1]
    pooled = (p0_ref[:G, :] + p1_ref[:G, :]) / jnp.maximum(cnt, 1.0)
    m = jnp.maximum(bn(jnp.dot(pooled, wm1_ref[...],
                               preferred_element_type=jnp.float32),
                       gm1_ref[...], bm1_ref[...]), 0.0)
    m = jnp.maximum(bn(jnp.dot(m, wm2_ref[...],
                               preferred_element_type=jnp.float32),
                       gm2_ref[...], bm2_ref[...]), 0.0)
    z = (jnp.dot(m, w2m_ref[...], preferred_element_type=jnp.float32) +
         jnp.dot(d_ref[...], w2d_ref[...], preferred_element_type=jnp.float32)
         + b2_ref[...])
    z = jnp.maximum(bn(z, g2_ref[...], be2_ref[...]), 0.0)
    o_ref[...] = jnp.dot(z, wo_ref[...],
                         preferred_element_type=jnp.float32) + bo_ref[...]


def _row_spec(w):
    return pl.BlockSpec((BLK, w), lambda i: (i, 0))


def _full(a):
    return pl.BlockSpec(a.shape, lambda i: tuple(0 for _ in a.shape))


_STAT_SPEC = pl.BlockSpec((1, HID), lambda i: (0, 0))


def kernel(x, edge_index, batch, descriptors, W1, b1, g1, be1, Wb_all, Wc_all,
           cb_all, ng_all, nb_all, Wm1, gm1, bm1, Wm2, gm2, bm2, W2, b2, g2,
           be2, Wo, bo):
    f32 = jnp.float32
    row = edge_index[0]
    col = edge_index[1]
    ones8 = jnp.ones((CH, 8), f32)
    z8 = jnp.zeros((STRIPE, 8), f32)
    z32 = jnp.zeros((STRIPE, 32), f32)
    zp = jnp.zeros((PSTRIPE, HID), f32)
    zc = jnp.zeros((PSTRIPE, 8), f32)
    batchp = jnp.concatenate(
        [batch, jnp.full((NPOOL - N,), G, jnp.int32)])

    d0, d1 = _sc_degree(col, ones8, z8)

    y, s, ss = pl.pallas_call(
        _lin1_body,
        grid=(GRID,),
        in_specs=[pl.BlockSpec((BLK, FIN), lambda i: (i, 0)),
                  _full(W1), pl.BlockSpec((1, HID), lambda i: (0, 0))],
        out_specs=[_row_spec(HID), _STAT_SPEC, _STAT_SPEC],
        out_shape=[jax.ShapeDtypeStruct((NPOOL, HID), f32),
                   jax.ShapeDtypeStruct((1, HID), f32),
                   jax.ShapeDtypeStruct((1, HID), f32)],
    )(x, W1, b1.reshape(1, HID))
    h = pl.pallas_call(
        _bn_body,
        grid=(GRID,),
        in_specs=[_row_spec(HID), _STAT_SPEC, _STAT_SPEC,
                  pl.BlockSpec((1, HID), lambda i: (0, 0)),
                  pl.BlockSpec((1, HID), lambda i: (0, 0))],
        out_specs=_row_spec(HID),
        out_shape=jax.ShapeDtypeStruct((NPOOL, HID), f32),
    )(y, s, ss, g1.reshape(1, HID), be1.reshape(1, HID))

    for l in range(LAYERS):
        lo, hi, comb = pl.pallas_call(
            _bc_body,
            grid=(GRID,),
            in_specs=[_row_spec(HID), _full(Wb_all[l]), _full(Wc_all[l]),
                      pl.BlockSpec((1, 32), lambda i: (0, 0)),
                      _row_spec(8), _row_spec(8)],
            out_specs=[_row_spec(32), _row_spec(32), _row_spec(32)],
            out_shape=[jax.ShapeDtypeStruct((N, 32), f32),
                       jax.ShapeDtypeStruct((N, 32), f32),
                       jax.ShapeDtypeStruct((N, 32), f32)],
        )(h, Wb_all[l], Wc_all[l], cb_all[l].reshape(1, 32), d0, d1)

        acclo, acchi = _sc_edge_agg(row, col, lo, hi, z32)

        y, s, ss = pl.pallas_call(
            _mix_body,
            grid=(GRID,),
            in_specs=[_row_spec(32), _row_spec(32), _row_spec(32),
                      _row_spec(32), _row_spec(32), _row_spec(8),
                      _row_spec(8)],
            out_specs=[_row_spec(HID), _STAT_SPEC, _STAT_SPEC],
            out_shape=[jax.ShapeDtypeStruct((NPOOL, HID), f32),
                       jax.ShapeDtypeStruct((1, HID), f32),
                       jax.ShapeDtypeStruct((1, HID), f32)],
        )(acclo, acchi, lo, hi, comb, d0, d1)

        h = pl.pallas_call(
            _bn_res_body,
            grid=(GRID,),
            in_specs=[_row_spec(HID), _STAT_SPEC, _STAT_SPEC,
                      pl.BlockSpec((1, HID), lambda i: (0, 0)),
                      pl.BlockSpec((1, HID), lambda i: (0, 0)),
                      _row_spec(HID)],
            out_specs=_row_spec(HID),
            out_shape=jax.ShapeDtypeStruct((NPOOL, HID), f32),
        )(y, s, ss, ng_all[l].reshape(1, HID), nb_all[l].reshape(1, HID), h)

    p0, p1, c0, c1 = _sc_pool(h, batchp, ones8, zp, zc)

    out = pl.pallas_call(
        _head_body,
        out_shape=jax.ShapeDtypeStruct((G, 1), f32),
    )(p0, p1, c0, c1, descriptors, Wm1, gm1.reshape(1, 64),
      bm1.reshape(1, 64), Wm2, gm2.reshape(1, 32), bm2.reshape(1, 32),
      W2[:32], W2[32:], b2.reshape(1, HID), g2.reshape(1, HID),
      be2.reshape(1, HID), Wo, bo.reshape(1, 1))
    return out
